# Initial kernel scaffold; baseline (speedup 1.0000x reference)
import jax
import jax.numpy as jnp
from jax.experimental import pallas as pl


def kernel(x0, x1, support_rows, support_cols, support_vals, support1_rows, support1_cols, support1_vals, total_mask, W0, W1, W00, W11, Wa0, Wa1):
    raise NotImplementedError("write your pallas kernel here")


# speedup vs baseline: 5.2233x; 5.2233x over previous
import functools

import jax
import jax.numpy as jnp
from jax import lax
from jax.experimental import pallas as pl
from jax.experimental.pallas import tpu as pltpu
from jax.experimental.pallas import tpu_sc as plsc

N = 10000
E = 160000
D = 256
HALF = 128
NTILE = 16
EPT = E // NTILE
EB = 50
NB = EPT // EB
CNB = 40
NCNK = NB // CNB
RB = 16
NCH = N // RB


def _lrelu(x):
    return jnp.maximum(x, 0.2 * x)


def _pre_body(x0_ref, w0_ref, x1_ref, w1_ref, p0_ref, p1_ref):
    p0_ref[0] = jnp.dot(x0_ref[...], w0_ref[...],
                        preferred_element_type=jnp.float32)
    p1_ref[0] = jnp.dot(x1_ref[...], w1_ref[...],
                        preferred_element_type=jnp.float32)


def _pre_matmuls(x0, W0, x1, W1):
    RBLK = 1000
    return pl.pallas_call(
        _pre_body,
        grid=(N // RBLK, 2),
        in_specs=[
            pl.BlockSpec((RBLK, D), lambda i, h: (i, 0)),
            pl.BlockSpec((D, HALF), lambda i, h: (0, h)),
            pl.BlockSpec((RBLK, D), lambda i, h: (i, 0)),
            pl.BlockSpec((D, HALF), lambda i, h: (0, h)),
        ],
        out_specs=[
            pl.BlockSpec((1, RBLK, HALF), lambda i, h: (h, i, 0)),
            pl.BlockSpec((1, RBLK, HALF), lambda i, h: (h, i, 0)),
        ],
        out_shape=[
            jax.ShapeDtypeStruct((2, N, HALF), jnp.float32),
            jax.ShapeDtypeStruct((2, N, HALF), jnp.float32),
        ],
    )(x0, W0, x1, W1)


def _spmm_body(pre0_hbm, rows0_hbm, cols0_hbm, vals0_hbm,
               pre1_hbm, rows1_hbm, cols1_hbm, vals1_hbm,
               out0_hbm, out1_hbm,
               rows_v, cols_v, vals_v, g0, g1, g2, g3, tbuf, zvec, acc,
               gs0, gs1, gs2, gs3, ss0, ss1, ss2, ss3):
    c = lax.axis_index("c")
    s = lax.axis_index("s")
    gbufs = (g0, g1, g2, g3)
    gsems = (gs0, gs1, gs2, gs3)
    ssems = (ss0, ss1, ss2, ss3)

    zvec[...] = jnp.zeros((16,), jnp.float32)

    def _zrow(r, _):
        for k in range(8):
            tbuf[r, pl.ds(k * 16, 16)] = zvec[...]
        return 0
    lax.fori_loop(0, RB, _zrow, 0)

    dnums = lax.GatherDimensionNumbers(
        offset_dims=(), collapsed_slice_dims=(0,), start_index_map=(0,))

    def _lane_bcast(vv, i):
        return lax.gather(vv, jnp.full((16, 1), i, jnp.int32), dnums,
                          slice_sizes=(1,),
                          mode=lax.GatherScatterMode.PROMISE_IN_BOUNDS)

    def _scale(j, b):
        gb = gbufs[b]
        base = j * EB

        def _group(g, _):
            e0 = g * 16
            vv = plsc.load_gather(
                vals_v,
                [jnp.full((16,), base + e0, jnp.int32)
                 + lax.iota(jnp.int32, 16)])
            for i in range(16):
                vb = _lane_bcast(vv, i)
                for k in range(8):
                    sl = pl.ds(k * 16, 16)
                    gb[e0 + i, sl] = gb[e0 + i, sl] * vb
            return 0
        lax.fori_loop(0, EB // 16, _group, 0)

        for e in range((EB // 16) * 16, EB):
            vb = plsc.load_gather(
                vals_v, [jnp.full((16,), base + e, jnp.int32)])
            for k in range(8):
                sl = pl.ds(k * 16, 16)
                gb[e, sl] = gb[e, sl] * vb

    def _support(pre_hbm, rows_hbm, cols_hbm, vals_hbm, out_hbm):
        def _zch(i, _):
            ch = s + i * NTILE
            @pl.when(ch < NCH)
            def _():
                pltpu.async_copy(tbuf, acc.at[pl.ds(ch * RB, RB)], gsems[0])
            return 0
        lax.fori_loop(0, (NCH + NTILE - 1) // NTILE, _zch, 0)

        def _zdrain(i, _):
            ch = s + i * NTILE
            @pl.when(ch < NCH)
            def _():
                pltpu.make_async_copy(
                    tbuf, acc.at[pl.ds(ch * RB, RB)], gsems[0]).wait()
            return 0
        lax.fori_loop(0, (NCH + NTILE - 1) // NTILE, _zdrain, 0)
        plsc.subcore_barrier()

        def _issue_gather(j, b):
            pltpu.async_copy(
                pre_hbm.at[c].at[cols_v.at[j]], gbufs[b], gsems[b])

        def _wait_gather(j, b):
            pltpu.make_async_copy(
                pre_hbm.at[c].at[cols_v.at[j]], gbufs[b], gsems[b]).wait()

        def _issue_scatter(j, b):
            pltpu.async_copy(
                gbufs[b], acc.at[rows_v.at[j]], ssems[b], add=True)

        def _wait_scatter(j, b):
            pltpu.make_async_copy(
                gbufs[b], acc.at[rows_v.at[j]], ssems[b]).wait()

        def _chunk(q, _):
            base = s * NB + q * CNB
            pltpu.sync_copy(rows_hbm.at[pl.ds(base, CNB)], rows_v)
            pltpu.sync_copy(cols_hbm.at[pl.ds(base, CNB)], cols_v)
            pltpu.sync_copy(vals_hbm.at[pl.ds(base * EB, CNB * EB)], vals_v)
            _issue_gather(0, 0)
            _issue_gather(1, 1)

            def _quad(p, _):
                for u in range(4):
                    j = 4 * p + u
                    _wait_gather(j, u)
                    _scale(j, u)
                    nxt = (u + 2) % 4
                    @pl.when(j >= 2)
                    def _():
                        _wait_scatter(jnp.maximum(j - 2, 0), nxt)
                    @pl.when(j + 2 < CNB)
                    def _():
                        _issue_gather(j + 2, nxt)
                    _issue_scatter(j, u)
                return 0
            lax.fori_loop(0, CNB // 4, _quad, 0)
            _wait_scatter(CNB - 2, 2)
            _wait_scatter(CNB - 1, 3)
            return 0
        lax.fori_loop(0, NCNK, _chunk, 0)
        plsc.subcore_barrier()

        WSP = 624
        WLAST = N - (NTILE - 1) * WSP

        @pl.when(s < NTILE - 1)
        def _():
            pltpu.sync_copy(
                acc.at[pl.ds(s * WSP, WSP)],
                out_hbm.at[pl.ds(s * WSP, WSP), pl.ds(c * HALF, HALF)])

        @pl.when(s == NTILE - 1)
        def _():
            pltpu.sync_copy(
                acc.at[pl.ds((NTILE - 1) * WSP, WLAST)],
                out_hbm.at[pl.ds((NTILE - 1) * WSP, WLAST),
                           pl.ds(c * HALF, HALF)])

    _support(pre0_hbm, rows0_hbm, cols0_hbm, vals0_hbm, out0_hbm)
    plsc.subcore_barrier()
    _support(pre1_hbm, rows1_hbm, cols1_hbm, vals1_hbm, out1_hbm)


def _spmm2(pre0, r0, c0, v0, pre1, r1, c1, v1):
    mesh = plsc.VectorSubcoreMesh(core_axis_name="c", subcore_axis_name="s")
    return pl.kernel(
        _spmm_body,
        out_type=[jax.ShapeDtypeStruct((N, D), jnp.float32),
                  jax.ShapeDtypeStruct((N, D), jnp.float32)],
        mesh=mesh,
        compiler_params=pltpu.CompilerParams(needs_layout_passes=False),
        scratch_types=[
            pltpu.VMEM((CNB, EB), jnp.int32),
            pltpu.VMEM((CNB, EB), jnp.int32),
            pltpu.VMEM((CNB * EB,), jnp.float32),
            pltpu.VMEM((EB, HALF), jnp.float32),
            pltpu.VMEM((EB, HALF), jnp.float32),
            pltpu.VMEM((EB, HALF), jnp.float32),
            pltpu.VMEM((EB, HALF), jnp.float32),
            pltpu.VMEM((RB, HALF), jnp.float32),
            pltpu.VMEM((16,), jnp.float32),
            pltpu.VMEM_SHARED((N, HALF), jnp.float32),
            pltpu.SemaphoreType.DMA,
            pltpu.SemaphoreType.DMA,
            pltpu.SemaphoreType.DMA,
            pltpu.SemaphoreType.DMA,
            pltpu.SemaphoreType.DMA,
            pltpu.SemaphoreType.DMA,
            pltpu.SemaphoreType.DMA,
            pltpu.SemaphoreType.DMA,
        ],
    )(pre0, r0, c0, v0, pre1, r1, c1, v1)


def _mix_body(i0_ref, i1_ref, w00_ref, w11_ref, m_ref, o0_ref, o1_ref):
    a0 = _lrelu(i0_ref[...])
    a1 = _lrelu(i1_ref[...])
    s0 = jnp.dot(a0, w00_ref[...], preferred_element_type=jnp.float32)
    s1 = jnp.dot(a1, w11_ref[...], preferred_element_type=jnp.float32)
    t = _lrelu(s0 + s1)
    m = m_ref[...]
    rm = 1.0 - m
    o0_ref[...] = a0 * rm + t * m
    o1_ref[...] = a1 * rm + t * m


def _mix(intra0, intra1, W00, W11, maskf):
    RBLK = 1000
    return pl.pallas_call(
        _mix_body,
        grid=(N // RBLK,),
        in_specs=[
            pl.BlockSpec((RBLK, D), lambda i: (i, 0)),
            pl.BlockSpec((RBLK, D), lambda i: (i, 0)),
            pl.BlockSpec((D, D), lambda i: (0, 0)),
            pl.BlockSpec((D, D), lambda i: (0, 0)),
            pl.BlockSpec((RBLK, 1), lambda i: (i, 0)),
        ],
        out_specs=[
            pl.BlockSpec((RBLK, D), lambda i: (i, 0)),
            pl.BlockSpec((RBLK, D), lambda i: (i, 0)),
        ],
        out_shape=[
            jax.ShapeDtypeStruct((N, D), jnp.float32),
            jax.ShapeDtypeStruct((N, D), jnp.float32),
        ],
    )(intra0, intra1, W00, W11, maskf)


@jax.jit
def kernel(x0, x1, support_rows, support_cols, support_vals,
           support1_rows, support1_cols, support1_vals,
           total_mask, W0, W1, W00, W11, Wa0, Wa1):
    del Wa0, Wa1
    r0 = support_rows.reshape(NTILE * NB, EB)
    c0 = support_cols.reshape(NTILE * NB, EB)
    r1 = support1_rows.reshape(NTILE * NB, EB)
    c1 = support1_cols.reshape(NTILE * NB, EB)

    pre0, pre1 = _pre_matmuls(x0, W0, x1, W1)
    intra0, intra1 = _spmm2(pre0, r0, c0, support_vals,
                            pre1, r1, c1, support1_vals)

    maskf = total_mask.astype(jnp.float32).reshape(N, 1)
    out0, out1 = _mix(intra0, intra1, W00, W11, maskf)
    return (out0, out1)


# speedup vs baseline: 5.3443x; 1.0232x over previous
import functools

import jax
import jax.numpy as jnp
from jax import lax
from jax.experimental import pallas as pl
from jax.experimental.pallas import tpu as pltpu
from jax.experimental.pallas import tpu_sc as plsc

N = 10000
E = 160000
D = 256
HALF = 128
NTILE = 16
EPT = E // NTILE
EB = 50
NB = EPT // EB
CNB = 40
NCNK = NB // CNB
RB = 16
NCH = N // RB


def _lrelu(x):
    return jnp.maximum(x, 0.2 * x)


def _pre_body(x_ref, w_ref, p_ref):
    p_ref[0] = jnp.dot(x_ref[...], w_ref[...],
                       preferred_element_type=jnp.float32)


def _pre_matmul(x, W):
    RBLK = 1000
    return pl.pallas_call(
        _pre_body,
        grid=(N // RBLK, 2),
        in_specs=[
            pl.BlockSpec((RBLK, D), lambda i, h: (i, 0)),
            pl.BlockSpec((D, HALF), lambda i, h: (0, h)),
        ],
        out_specs=pl.BlockSpec((1, RBLK, HALF), lambda i, h: (h, i, 0)),
        out_shape=jax.ShapeDtypeStruct((2, N, HALF), jnp.float32),
    )(x, W)


def _spmm_body(pre_hbm, rows_hbm, cols_hbm, vals_hbm, out_hbm,
               rows_v, cols_v, vals_v, g0, g1, g2, g3, tbuf, zvec, acc,
               gs0, gs1, gs2, gs3, ss0, ss1, ss2, ss3):
    c = lax.axis_index("c")
    s = lax.axis_index("s")
    gbufs = (g0, g1, g2, g3)
    gsems = (gs0, gs1, gs2, gs3)
    ssems = (ss0, ss1, ss2, ss3)

    zvec[...] = jnp.zeros((16,), jnp.float32)

    def _zrow(r, _):
        for k in range(8):
            tbuf[r, pl.ds(k * 16, 16)] = zvec[...]
        return 0
    lax.fori_loop(0, RB, _zrow, 0)

    def _zch(i, _):
        ch = s + i * NTILE
        @pl.when(ch < NCH)
        def _():
            pltpu.async_copy(tbuf, acc.at[pl.ds(ch * RB, RB)], gsems[0])
        return 0
    lax.fori_loop(0, (NCH + NTILE - 1) // NTILE, _zch, 0)

    def _zdrain(i, _):
        ch = s + i * NTILE
        @pl.when(ch < NCH)
        def _():
            pltpu.make_async_copy(
                tbuf, acc.at[pl.ds(ch * RB, RB)], gsems[0]).wait()
        return 0
    lax.fori_loop(0, (NCH + NTILE - 1) // NTILE, _zdrain, 0)
    plsc.subcore_barrier()

    def _issue_gather(j, b):
        pltpu.async_copy(pre_hbm.at[c].at[cols_v.at[j]], gbufs[b], gsems[b])

    def _wait_gather(j, b):
        pltpu.make_async_copy(
            pre_hbm.at[c].at[cols_v.at[j]], gbufs[b], gsems[b]).wait()

    def _issue_scatter(j, b):
        pltpu.async_copy(gbufs[b], acc.at[rows_v.at[j]], ssems[b], add=True)

    def _wait_scatter(j, b):
        pltpu.make_async_copy(
            gbufs[b], acc.at[rows_v.at[j]], ssems[b]).wait()

    dnums = lax.GatherDimensionNumbers(
        offset_dims=(), collapsed_slice_dims=(0,), start_index_map=(0,))

    def _lane_bcast(vv, i):
        return lax.gather(vv, jnp.full((16, 1), i, jnp.int32), dnums,
                          slice_sizes=(1,),
                          mode=lax.GatherScatterMode.PROMISE_IN_BOUNDS)

    def _scale(j, b):
        gb = gbufs[b]
        base = j * EB

        def _group(g, _):
            e0 = g * 16
            vv = plsc.load_gather(
                vals_v,
                [jnp.full((16,), base + e0, jnp.int32)
                 + lax.iota(jnp.int32, 16)])
            for i in range(16):
                vb = _lane_bcast(vv, i)
                for k in range(8):
                    sl = pl.ds(k * 16, 16)
                    gb[e0 + i, sl] = gb[e0 + i, sl] * vb
            return 0
        lax.fori_loop(0, EB // 16, _group, 0)

        for e in range((EB // 16) * 16, EB):
            vb = plsc.load_gather(
                vals_v, [jnp.full((16,), base + e, jnp.int32)])
            for k in range(8):
                sl = pl.ds(k * 16, 16)
                gb[e, sl] = gb[e, sl] * vb

    def _chunk(q, _):
        base = s * NB + q * CNB
        pltpu.sync_copy(rows_hbm.at[pl.ds(base, CNB)], rows_v)
        pltpu.sync_copy(cols_hbm.at[pl.ds(base, CNB)], cols_v)
        pltpu.sync_copy(vals_hbm.at[pl.ds(base * EB, CNB * EB)], vals_v)
        _issue_gather(0, 0)
        _issue_gather(1, 1)

        def _quad(p, _):
            for u in range(4):
                j = 4 * p + u
                _wait_gather(j, u)
                _scale(j, u)
                nxt = (u + 2) % 4
                @pl.when(j >= 2)
                def _():
                    _wait_scatter(jnp.maximum(j - 2, 0), nxt)
                @pl.when(j + 2 < CNB)
                def _():
                    _issue_gather(j + 2, nxt)
                _issue_scatter(j, u)
            return 0
        lax.fori_loop(0, CNB // 4, _quad, 0)
        _wait_scatter(CNB - 2, 2)
        _wait_scatter(CNB - 1, 3)
        return 0
    lax.fori_loop(0, NCNK, _chunk, 0)
    plsc.subcore_barrier()

    WSP = 624
    WLAST = N - (NTILE - 1) * WSP

    @pl.when(s < NTILE - 1)
    def _():
        pltpu.sync_copy(
            acc.at[pl.ds(s * WSP, WSP)],
            out_hbm.at[pl.ds(s * WSP, WSP), pl.ds(c * HALF, HALF)])

    @pl.when(s == NTILE - 1)
    def _():
        pltpu.sync_copy(
            acc.at[pl.ds((NTILE - 1) * WSP, WLAST)],
            out_hbm.at[pl.ds((NTILE - 1) * WSP, WLAST),
                       pl.ds(c * HALF, HALF)])


def _spmm(pre, rows2d, cols2d, vals2d):
    mesh = plsc.VectorSubcoreMesh(core_axis_name="c", subcore_axis_name="s")
    return pl.kernel(
        _spmm_body,
        out_type=jax.ShapeDtypeStruct((N, D), jnp.float32),
        mesh=mesh,
        compiler_params=pltpu.CompilerParams(needs_layout_passes=False),
        scratch_types=[
            pltpu.VMEM((CNB, EB), jnp.int32),
            pltpu.VMEM((CNB, EB), jnp.int32),
            pltpu.VMEM((CNB * EB,), jnp.float32),
            pltpu.VMEM((EB, HALF), jnp.float32),
            pltpu.VMEM((EB, HALF), jnp.float32),
            pltpu.VMEM((EB, HALF), jnp.float32),
            pltpu.VMEM((EB, HALF), jnp.float32),
            pltpu.VMEM((RB, HALF), jnp.float32),
            pltpu.VMEM((16,), jnp.float32),
            pltpu.VMEM_SHARED((N, HALF), jnp.float32),
            pltpu.SemaphoreType.DMA,
            pltpu.SemaphoreType.DMA,
            pltpu.SemaphoreType.DMA,
            pltpu.SemaphoreType.DMA,
            pltpu.SemaphoreType.DMA,
            pltpu.SemaphoreType.DMA,
            pltpu.SemaphoreType.DMA,
            pltpu.SemaphoreType.DMA,
        ],
    )(pre, rows2d, cols2d, vals2d)


def _mix_body(i0_ref, i1_ref, w00_ref, w11_ref, m_ref, o0_ref, o1_ref):
    a0 = _lrelu(i0_ref[...])
    a1 = _lrelu(i1_ref[...])
    s0 = jnp.dot(a0, w00_ref[...], preferred_element_type=jnp.float32)
    s1 = jnp.dot(a1, w11_ref[...], preferred_element_type=jnp.float32)
    t = _lrelu(s0 + s1)
    m = m_ref[...]
    rm = 1.0 - m
    o0_ref[...] = a0 * rm + t * m
    o1_ref[...] = a1 * rm + t * m


def _mix(intra0, intra1, W00, W11, maskf):
    RBLK = 1000
    return pl.pallas_call(
        _mix_body,
        grid=(N // RBLK,),
        in_specs=[
            pl.BlockSpec((RBLK, D), lambda i: (i, 0)),
            pl.BlockSpec((RBLK, D), lambda i: (i, 0)),
            pl.BlockSpec((D, D), lambda i: (0, 0)),
            pl.BlockSpec((D, D), lambda i: (0, 0)),
            pl.BlockSpec((RBLK, 1), lambda i: (i, 0)),
        ],
        out_specs=[
            pl.BlockSpec((RBLK, D), lambda i: (i, 0)),
            pl.BlockSpec((RBLK, D), lambda i: (i, 0)),
        ],
        out_shape=[
            jax.ShapeDtypeStruct((N, D), jnp.float32),
            jax.ShapeDtypeStruct((N, D), jnp.float32),
        ],
    )(intra0, intra1, W00, W11, maskf)


@jax.jit
def kernel(x0, x1, support_rows, support_cols, support_vals,
           support1_rows, support1_cols, support1_vals,
           total_mask, W0, W1, W00, W11, Wa0, Wa1):
    del Wa0, Wa1
    r0 = support_rows.reshape(NTILE * NB, EB)
    c0 = support_cols.reshape(NTILE * NB, EB)
    r1 = support1_rows.reshape(NTILE * NB, EB)
    c1 = support1_cols.reshape(NTILE * NB, EB)

    pre0 = _pre_matmul(x0, W0)
    intra0 = _spmm(pre0, r0, c0, support_vals)
    pre1 = _pre_matmul(x1, W1)
    intra1 = _spmm(pre1, r1, c1, support1_vals)

    maskf = total_mask.astype(jnp.float32).reshape(N, 1)
    out0, out1 = _mix(intra0, intra1, W00, W11, maskf)
    return (out0, out1)
